# baseline (speedup 1.0000x reference)
import jax
import jax.numpy as jnp
from jax import lax
from jax.experimental import pallas as pl
from jax.experimental.pallas import tpu as pltpu
from jax.experimental.pallas import tpu_sc as plsc

_S = 2048
_D = 768
_RC = 2568
_PAD = 2560
_NW = 32
_SPW = 25
_SLOTS = 28
_G = 7
_GS = 4
_KR = 6
_KC = 4


def _build_body(ctx_ref, comb_ref):
    x = ctx_ref[0]
    r = x.reshape(512, 4, _D)
    comb_ref[0, 0:512, :] = jnp.maximum(
        jnp.maximum(r[:, 0, :], r[:, 1, :]),
        jnp.maximum(r[:, 2, :], r[:, 3, :]))
    for k in range(1, 5):
        s = 4 ** (k - 1)
        n = 512 - 3 * s
        prev = (k - 1) * 512
        a = comb_ref[0, pl.ds(prev, n), :]
        b = comb_ref[0, pl.ds(prev + s, n), :]
        c = comb_ref[0, pl.ds(prev + 2 * s, n), :]
        d = comb_ref[0, pl.ds(prev + 3 * s, n), :]
        comb_ref[0, pl.ds(k * 512, n), :] = jnp.maximum(
            jnp.maximum(a, b), jnp.maximum(c, d))
    comb_ref[0, pl.ds(_PAD, 8), :] = jnp.full((8, _D), -jnp.inf, jnp.float32)


def _sc_query(ctx_hbm, comb_hbm, idxr_hbm, idxc_hbm, out_hbm,
              idxr_v, idxc_v, raw0, raw1, cmb0, cmb1, out_v,
              sr0, sr1, sc0, sc1):
    w = lax.axis_index("s") * 2 + lax.axis_index("c")
    pltpu.sync_copy(idxr_hbm.at[w], idxr_v)
    pltpu.sync_copy(idxc_hbm.at[w], idxc_v)
    raws, cmbs = [raw0, raw1], [cmb0, cmb1]
    srs, scs = [sr0, sr1], [sc0, sc1]
    hr, hc = [None] * _G, [None] * _G

    def issue(g):
        hr[g] = pltpu.async_copy(ctx_hbm.at[idxr_v.at[g]], raws[g % 2],
                                 srs[g % 2])
        hc[g] = pltpu.async_copy(comb_hbm.at[idxc_v.at[g]], cmbs[g % 2],
                                 scs[g % 2])

    issue(0)
    for g in range(_G):
        if g + 1 < _G:
            issue(g + 1)
        hr[g].wait()
        hc[g].wait()
        rb, cb = raws[g % 2], cmbs[g % 2]
        for s4 in range(_GS):
            def chunk(c, _, s4=s4, rb=rb, cb=cb, g=g):
                for u in range(3):
                    off = c * 48 + u * 16
                    acc = rb[s4 * _KR, pl.ds(off, 16)]
                    for r in range(1, _KR):
                        acc = jnp.maximum(acc,
                                          rb[s4 * _KR + r, pl.ds(off, 16)])
                    for r in range(_KC):
                        acc = jnp.maximum(acc,
                                          cb[s4 * _KC + r, pl.ds(off, 16)])
                    out_v[g * _GS + s4, pl.ds(off, 16)] = acc
                return 0
            jax.lax.fori_loop(0, _D // 48, chunk, 0)
    pltpu.sync_copy(out_v, out_hbm.at[pl.ds(w * 32, 32)])


def _make_indices(spans, B, S, n):
    b = spans[..., 0].astype(jnp.int32).reshape(-1)
    e = spans[..., 1].astype(jnp.int32).reshape(-1)
    e = jnp.where(e == b, b + 1, e)

    a1 = (b + 3) & ~3
    z1 = e & ~3
    w4 = a1 > z1
    nl = jnp.where(w4, e - b, a1 - b)
    nr = jnp.where(w4, 0, e - z1)
    jr = jnp.arange(_KR, dtype=jnp.int32)[None, :]
    raw = jnp.where(jr < nl[:, None], b[:, None] + jr,
                    jnp.where(jr < (nl + nr)[:, None],
                              z1[:, None] + jr - nl[:, None], b[:, None]))

    p = a1 >> 2
    q = z1 >> 2
    il = q - p
    hast = (~w4) & (il > 0)
    k4 = (31 - lax.clz(jnp.maximum(il, 1))) >> 1
    s4 = jnp.int32(1) << (2 * k4)
    ji = jnp.arange(_KC, dtype=jnp.int32)[None, :]
    cov = jnp.minimum(p[:, None] + ji * s4[:, None], (q - s4)[:, None])
    comb = jnp.where(hast[:, None], (k4 << 9)[:, None] + cov, _PAD)

    boff = (jnp.arange(B * n, dtype=jnp.int32) // n)[:, None]
    raw = (raw + boff * S).reshape(_NW, _SPW, _KR)
    comb = (comb + boff * _RC).reshape(_NW, _SPW, _KC)
    idx_raw = jnp.pad(raw, ((0, 0), (0, _SLOTS - _SPW), (0, 0))
                      ).reshape(_NW, _G, _GS * _KR)
    idx_cmb = jnp.pad(comb, ((0, 0), (0, _SLOTS - _SPW), (0, 0)),
                      constant_values=_PAD).reshape(_NW, _G, _GS * _KC)
    return idx_raw, idx_cmb


@jax.jit
def kernel(context, spans):
    B, S, D = context.shape
    n = spans.shape[1]
    idx_raw, idx_cmb = _make_indices(spans, B, S, n)

    comb_rows = pl.pallas_call(
        _build_body,
        grid=(B,),
        in_specs=[pl.BlockSpec((1, S, D), lambda i: (i, 0, 0))],
        out_specs=pl.BlockSpec((1, _RC, D), lambda i: (i, 0, 0)),
        out_shape=jax.ShapeDtypeStruct((B, _RC, D), jnp.float32),
    )(context)

    mesh = plsc.VectorSubcoreMesh(core_axis_name="c", subcore_axis_name="s")
    out_pad = pl.kernel(
        _sc_query,
        mesh=mesh,
        out_type=jax.ShapeDtypeStruct((_NW * 32, D), jnp.float32),
        scratch_types=[
            pltpu.VMEM((_G, _GS * _KR), jnp.int32),
            pltpu.VMEM((_G, _GS * _KC), jnp.int32),
            pltpu.VMEM((_GS * _KR, D), jnp.float32),
            pltpu.VMEM((_GS * _KR, D), jnp.float32),
            pltpu.VMEM((_GS * _KC, D), jnp.float32),
            pltpu.VMEM((_GS * _KC, D), jnp.float32),
            pltpu.VMEM((32, D), jnp.float32),
            pltpu.SemaphoreType.DMA,
            pltpu.SemaphoreType.DMA,
            pltpu.SemaphoreType.DMA,
            pltpu.SemaphoreType.DMA,
        ],
    )(context.reshape(B * S, D), comb_rows.reshape(B * _RC, D),
      idx_raw, idx_cmb)

    return out_pad.reshape(_NW, 32, D)[:, :_SPW, :].reshape(B, n, D)


# speedup vs baseline: 1.0413x; 1.0413x over previous
import jax
import jax.numpy as jnp
from jax import lax
from jax.experimental import pallas as pl
from jax.experimental.pallas import tpu as pltpu
from jax.experimental.pallas import tpu_sc as plsc

_S = 2048
_D = 768
_RC = 2568
_PAD = 2560
_NW = 32
_SPW = 25
_SLOTS = 28
_G = 7
_GS = 4
_KR = 6
_KC = 4


def _build_body(ctx_ref, comb_ref):
    x = ctx_ref[0]
    r = x.reshape(512, 4, _D)
    comb_ref[0, 0:512, :] = jnp.maximum(
        jnp.maximum(r[:, 0, :], r[:, 1, :]),
        jnp.maximum(r[:, 2, :], r[:, 3, :]))
    for k in range(1, 5):
        s = 4 ** (k - 1)
        n = 512 - 3 * s
        prev = (k - 1) * 512
        a = comb_ref[0, pl.ds(prev, n), :]
        b = comb_ref[0, pl.ds(prev + s, n), :]
        c = comb_ref[0, pl.ds(prev + 2 * s, n), :]
        d = comb_ref[0, pl.ds(prev + 3 * s, n), :]
        comb_ref[0, pl.ds(k * 512, n), :] = jnp.maximum(
            jnp.maximum(a, b), jnp.maximum(c, d))
    comb_ref[0, pl.ds(_PAD, 8), :] = jnp.full((8, _D), -jnp.inf, jnp.float32)


def _sc_query(ctx_hbm, comb_hbm, idxr_hbm, idxc_hbm, out_hbm,
              idxr_v, idxc_v, raw0, raw1, raw2, cmb0, cmb1, cmb2, out_v,
              sr0, sr1, sr2, sc0, sc1, sc2):
    w = lax.axis_index("s") * 2 + lax.axis_index("c")
    pltpu.sync_copy(idxr_hbm.at[w], idxr_v)
    pltpu.sync_copy(idxc_hbm.at[w], idxc_v)
    raws, cmbs = [raw0, raw1, raw2], [cmb0, cmb1, cmb2]
    srs, scs = [sr0, sr1, sr2], [sc0, sc1, sc2]
    hr, hc = [None] * _G, [None] * _G

    def issue(g):
        hr[g] = pltpu.async_copy(ctx_hbm.at[idxr_v.at[g]], raws[g % 3],
                                 srs[g % 3])
        hc[g] = pltpu.async_copy(comb_hbm.at[idxc_v.at[g]], cmbs[g % 3],
                                 scs[g % 3])

    issue(0)
    issue(1)
    for g in range(_G):
        if g + 2 < _G:
            issue(g + 2)
        hr[g].wait()
        hc[g].wait()
        rb, cb = raws[g % 3], cmbs[g % 3]
        for s4 in range(_GS):
            def chunk(c, _, s4=s4, rb=rb, cb=cb, g=g):
                off = c * 16
                acc = rb[s4 * _KR, pl.ds(off, 16)]
                for r in range(1, _KR):
                    acc = jnp.maximum(acc, rb[s4 * _KR + r, pl.ds(off, 16)])
                for r in range(_KC):
                    acc = jnp.maximum(acc, cb[s4 * _KC + r, pl.ds(off, 16)])
                out_v[g * _GS + s4, pl.ds(off, 16)] = acc
                return 0
            jax.lax.fori_loop(0, _D // 16, chunk, 0)
    pltpu.sync_copy(out_v, out_hbm.at[pl.ds(w * 32, 32)])


def _make_indices(spans, B, S, n):
    b = spans[..., 0].astype(jnp.int32).reshape(-1)
    e = spans[..., 1].astype(jnp.int32).reshape(-1)
    e = jnp.where(e == b, b + 1, e)

    a1 = (b + 3) & ~3
    z1 = e & ~3
    w4 = a1 > z1
    nl = jnp.where(w4, e - b, a1 - b)
    nr = jnp.where(w4, 0, e - z1)
    jr = jnp.arange(_KR, dtype=jnp.int32)[None, :]
    raw = jnp.where(jr < nl[:, None], b[:, None] + jr,
                    jnp.where(jr < (nl + nr)[:, None],
                              z1[:, None] + jr - nl[:, None], b[:, None]))

    p = a1 >> 2
    q = z1 >> 2
    il = q - p
    hast = (~w4) & (il > 0)
    k4 = (31 - lax.clz(jnp.maximum(il, 1))) >> 1
    s4 = jnp.int32(1) << (2 * k4)
    ji = jnp.arange(_KC, dtype=jnp.int32)[None, :]
    cov = jnp.minimum(p[:, None] + ji * s4[:, None], (q - s4)[:, None])
    comb = jnp.where(hast[:, None], (k4 << 9)[:, None] + cov, _PAD)

    boff = (jnp.arange(B * n, dtype=jnp.int32) // n)[:, None]
    raw = (raw + boff * S).reshape(_NW, _SPW, _KR)
    comb = (comb + boff * _RC).reshape(_NW, _SPW, _KC)
    idx_raw = jnp.pad(raw, ((0, 0), (0, _SLOTS - _SPW), (0, 0))
                      ).reshape(_NW, _G, _GS * _KR)
    idx_cmb = jnp.pad(comb, ((0, 0), (0, _SLOTS - _SPW), (0, 0)),
                      constant_values=_PAD).reshape(_NW, _G, _GS * _KC)
    return idx_raw, idx_cmb


@jax.jit
def kernel(context, spans):
    B, S, D = context.shape
    n = spans.shape[1]
    idx_raw, idx_cmb = _make_indices(spans, B, S, n)

    comb_rows = pl.pallas_call(
        _build_body,
        grid=(B,),
        in_specs=[pl.BlockSpec((1, S, D), lambda i: (i, 0, 0))],
        out_specs=pl.BlockSpec((1, _RC, D), lambda i: (i, 0, 0)),
        out_shape=jax.ShapeDtypeStruct((B, _RC, D), jnp.float32),
    )(context)

    mesh = plsc.VectorSubcoreMesh(core_axis_name="c", subcore_axis_name="s")
    out_pad = pl.kernel(
        _sc_query,
        mesh=mesh,
        out_type=jax.ShapeDtypeStruct((_NW * 32, D), jnp.float32),
        scratch_types=[
            pltpu.VMEM((_G, _GS * _KR), jnp.int32),
            pltpu.VMEM((_G, _GS * _KC), jnp.int32),
            pltpu.VMEM((_GS * _KR, D), jnp.float32),
            pltpu.VMEM((_GS * _KR, D), jnp.float32),
            pltpu.VMEM((_GS * _KR, D), jnp.float32),
            pltpu.VMEM((_GS * _KC, D), jnp.float32),
            pltpu.VMEM((_GS * _KC, D), jnp.float32),
            pltpu.VMEM((_GS * _KC, D), jnp.float32),
            pltpu.VMEM((32, D), jnp.float32),
            pltpu.SemaphoreType.DMA,
            pltpu.SemaphoreType.DMA,
            pltpu.SemaphoreType.DMA,
            pltpu.SemaphoreType.DMA,
            pltpu.SemaphoreType.DMA,
            pltpu.SemaphoreType.DMA,
        ],
    )(context.reshape(B * S, D), comb_rows.reshape(B * _RC, D),
      idx_raw, idx_cmb)

    return out_pad.reshape(_NW, 32, D)[:, :_SPW, :].reshape(B, n, D)
